# baseline (speedup 1.0000x reference)
import functools
import math

import jax
import jax.numpy as jnp
from jax import lax
from jax.experimental import pallas as pl
from jax.experimental.pallas import tpu as pltpu
from jax.experimental.pallas import tpu_sc as plsc

_SCALE = 64.0
_MARGIN = 0.5
_COS_M = math.cos(_MARGIN)
_SIN_M = math.sin(_MARGIN)
_THETA = math.cos(math.pi - _MARGIN)
_SINMM = math.sin(math.pi - _MARGIN) * _MARGIN

_B = 4096
_C0 = 100
_C1 = 1000
_D = 512
_BS = 32
_NB = 232
_NPAD = _NB * _BS
_NW = 32
_BPW = _NPAD // _NW
_CH0, _CH1 = 120, 112
_EPS = 1e-12


def _routing(labels):
    labels = labels.astype(jnp.int32)
    group = labels // _C1
    local = labels % _C1
    order = jnp.argsort(group).astype(jnp.int32)
    counts = jnp.bincount(group, length=_C0).astype(jnp.int32)
    bpg = (counts + _BS - 1) // _BS
    cum_bpg = jnp.cumsum(bpg)
    total_blocks = cum_bpg[_C0 - 1]
    bidx = jnp.arange(_NB, dtype=jnp.int32)
    raw = jnp.searchsorted(cum_bpg, bidx, side="right").astype(jnp.int32)
    last_gid = group[order[_B - 1]]
    block_gid = jnp.where(bidx < total_blocks, jnp.minimum(raw, _C0 - 1), last_gid)
    block_start = cum_bpg - bpg
    group_start = jnp.cumsum(counts) - counts
    p = jnp.arange(_NPAD, dtype=jnp.int32)
    b = p // _BS
    g = block_gid[b]
    r = (b - block_start[g]) * _BS + (p % _BS)
    valid = (r < counts[g]) & (b < total_blocks)
    src = jnp.where(valid, order[jnp.clip(group_start[g] + r, 0, _B - 1)], 0)
    src = src.astype(jnp.int32)
    rloc = jnp.where(valid, local[src], 0).astype(jnp.int32)
    return block_gid, src, rloc, valid


def _sc_gather(logits, src):
    mesh = plsc.VectorSubcoreMesh(core_axis_name="c", subcore_axis_name="s")

    @functools.partial(
        pl.kernel,
        out_type=jax.ShapeDtypeStruct((_NPAD, _D), jnp.float32),
        mesh=mesh,
        scratch_types=[
            pltpu.VMEM((_CH0,), jnp.int32),
            pltpu.VMEM((_CH1,), jnp.int32),
            pltpu.VMEM((_CH0, _D), jnp.float32),
            pltpu.VMEM((_CH1, _D), jnp.float32),
            pltpu.SemaphoreType.DMA,
            pltpu.SemaphoreType.DMA,
        ],
    )
    def k(logits_hbm, src_hbm, out_hbm, idx_a, idx_b, rows_a, rows_b, sem_a, sem_b):
        wid = lax.axis_index("s") * 2 + lax.axis_index("c")
        base = wid * _BPW
        pltpu.sync_copy(src_hbm.at[pl.ds(base, _CH0)], idx_a)
        pltpu.sync_copy(src_hbm.at[pl.ds(base + _CH0, _CH1)], idx_b)
        ga = pltpu.async_copy(logits_hbm.at[idx_a], rows_a, sem_a)
        gb = pltpu.async_copy(logits_hbm.at[idx_b], rows_b, sem_b)
        ga.wait()
        wa = pltpu.async_copy(rows_a, out_hbm.at[pl.ds(base, _CH0)], sem_a)
        gb.wait()
        wb = pltpu.async_copy(rows_b, out_hbm.at[pl.ds(base + _CH0, _CH1)], sem_b)
        wa.wait()
        wb.wait()

    return k(logits, src)


def _arc_step(gid_ref, xg_ref, w_ref, loc_ref, val_ref, out_ref, winv_ref):
    i = pl.program_id(0)
    g = gid_ref[i]
    prev = gid_ref[jnp.maximum(i - 1, 0)]
    w = w_ref[0]

    @pl.when((i == 0) | (g != prev))
    def _():
        ss = jnp.sum(w * w, axis=1)
        winv_ref[...] = (1.0 / jnp.maximum(jnp.sqrt(ss), _EPS))[None, :]

    x = xg_ref[...]
    xinv = 1.0 / jnp.maximum(
        jnp.sqrt(jnp.sum(x * x, axis=1, keepdims=True)), _EPS)
    xh = x * xinv
    dot = lax.dot_general(xh, w, (((1,), (1,)), ((), ())),
                          preferred_element_type=jnp.float32)
    cos = dot * winv_ref[...]
    loc = loc_ref[0]
    val = val_ref[0]
    cols = lax.broadcasted_iota(jnp.int32, (_BS, _C1), 1)
    mask = cols == loc
    t = jnp.sum(jnp.where(mask, cos, 0.0), axis=1, keepdims=True)
    sin_t = jnp.sqrt(jnp.clip(1.0 - t * t, 0.0, None))
    ctm = t * _COS_M - sin_t * _SIN_M
    ft = jnp.where(t > _THETA, ctm, t - _SINMM)
    cos2 = jnp.where(mask, ft, cos)
    s = cos2 * _SCALE
    m = jnp.max(s, axis=1, keepdims=True)
    lse = m + jnp.log(jnp.sum(jnp.exp(s - m), axis=1, keepdims=True))
    part = jnp.sum((lse - ft * _SCALE) * val) * (1.0 / _B)

    @pl.when(i == 0)
    def _():
        out_ref[...] = jnp.zeros((1, 1), jnp.float32)

    out_ref[...] = out_ref[...] + part


def _tc_grid_spec():
    return pltpu.PrefetchScalarGridSpec(
        num_scalar_prefetch=1,
        grid=(_NB,),
        in_specs=[
            pl.BlockSpec((_BS, _D), lambda i, gid: (i, 0)),
            pl.BlockSpec((1, _C1, _D), lambda i, gid: (gid[i], 0, 0)),
            pl.BlockSpec((1, _BS, 1), lambda i, gid: (i, 0, 0)),
            pl.BlockSpec((1, _BS, 1), lambda i, gid: (i, 0, 0)),
        ],
        out_specs=pl.BlockSpec((1, 1), lambda i, gid: (0, 0)),
        scratch_shapes=[pltpu.VMEM((1, _C1), jnp.float32)],
    )


def kernel(logits, labels, weight):
    block_gid, src, rloc, valid = _routing(labels)
    xg = _sc_gather(logits, src)
    loc3 = rloc.reshape(_NB, _BS, 1)
    val3 = valid.astype(jnp.float32).reshape(_NB, _BS, 1)
    out = pl.pallas_call(
        _arc_step,
        grid_spec=_tc_grid_spec(),
        out_shape=jax.ShapeDtypeStruct((1, 1), jnp.float32),
    )(block_gid, xg, weight, loc3, val3)
    return out[0, 0]


# speedup vs baseline: 1.0574x; 1.0574x over previous
import functools
import math

import jax
import jax.numpy as jnp
from jax import lax
from jax.experimental import pallas as pl
from jax.experimental.pallas import tpu as pltpu
from jax.experimental.pallas import tpu_sc as plsc

_SCALE = 64.0
_MARGIN = 0.5
_COS_M = math.cos(_MARGIN)
_SIN_M = math.sin(_MARGIN)
_THETA = math.cos(math.pi - _MARGIN)
_SINMM = math.sin(math.pi - _MARGIN) * _MARGIN

_B = 4096
_C0 = 100
_C1 = 1000
_D = 512
_BS = 32
_NB = 232
_NPAD = _NB * _BS
_NW = 32
_BPW = _NPAD // _NW
_CH0, _CH1 = 120, 112
_EPS = 1e-12


def _routing(labels):
    labels = labels.astype(jnp.int32)
    group = labels // _C1
    local = labels % _C1
    order = jnp.argsort(group).astype(jnp.int32)
    counts = jnp.bincount(group, length=_C0).astype(jnp.int32)
    bpg = (counts + _BS - 1) // _BS
    cum_bpg = jnp.cumsum(bpg)
    total_blocks = cum_bpg[_C0 - 1]
    bidx = jnp.arange(_NB, dtype=jnp.int32)
    raw = jnp.searchsorted(cum_bpg, bidx, side="right").astype(jnp.int32)
    last_gid = group[order[_B - 1]]
    block_gid = jnp.where(bidx < total_blocks, jnp.minimum(raw, _C0 - 1), last_gid)
    block_start = cum_bpg - bpg
    group_start = jnp.cumsum(counts) - counts
    p = jnp.arange(_NPAD, dtype=jnp.int32)
    b = p // _BS
    g = block_gid[b]
    r = (b - block_start[g]) * _BS + (p % _BS)
    valid = (r < counts[g]) & (b < total_blocks)
    src = jnp.where(valid, order[jnp.clip(group_start[g] + r, 0, _B - 1)], 0)
    src = src.astype(jnp.int32)
    rloc = jnp.where(valid, local[src], 0).astype(jnp.int32)
    return block_gid, src, rloc, valid


def _sc_gather(logits, src):
    mesh = plsc.VectorSubcoreMesh(core_axis_name="c", subcore_axis_name="s")

    @functools.partial(
        pl.kernel,
        out_type=jax.ShapeDtypeStruct((_NPAD, _D), jnp.float32),
        mesh=mesh,
        scratch_types=[
            pltpu.VMEM((_CH0,), jnp.int32),
            pltpu.VMEM((_CH1,), jnp.int32),
            pltpu.VMEM((_CH0, _D), jnp.float32),
            pltpu.VMEM((_CH1, _D), jnp.float32),
            pltpu.SemaphoreType.DMA,
            pltpu.SemaphoreType.DMA,
        ],
    )
    def k(logits_hbm, src_hbm, out_hbm, idx_a, idx_b, rows_a, rows_b, sem_a, sem_b):
        wid = lax.axis_index("s") * 2 + lax.axis_index("c")
        base = wid * _BPW
        pltpu.sync_copy(src_hbm.at[pl.ds(base, _CH0)], idx_a)
        pltpu.sync_copy(src_hbm.at[pl.ds(base + _CH0, _CH1)], idx_b)
        ga = pltpu.async_copy(logits_hbm.at[idx_a], rows_a, sem_a)
        gb = pltpu.async_copy(logits_hbm.at[idx_b], rows_b, sem_b)
        ga.wait()
        wa = pltpu.async_copy(rows_a, out_hbm.at[pl.ds(base, _CH0)], sem_a)
        gb.wait()
        wb = pltpu.async_copy(rows_b, out_hbm.at[pl.ds(base + _CH0, _CH1)], sem_b)
        wa.wait()
        wb.wait()

    return k(logits, src)


def _arc_step(gid_ref, xg_ref, w_ref, loc_ref, val_ref, out_ref, winv_ref):
    i = pl.program_id(0)
    g = gid_ref[i]
    prev = gid_ref[jnp.maximum(i - 1, 0)]
    w = w_ref[0]

    @pl.when((i == 0) | (g != prev))
    def _():
        ss = jnp.sum(w * w, axis=1, keepdims=True)
        winv_ref[...] = _SCALE / jnp.maximum(jnp.sqrt(ss), _EPS)

    x = xg_ref[...]
    xinv = 1.0 / jnp.maximum(
        jnp.sqrt(jnp.sum(x * x, axis=1, keepdims=True)), _EPS)
    xh = x * xinv
    dot = lax.dot_general(w, xh, (((1,), (1,)), ((), ())),
                          preferred_element_type=jnp.float32)
    s = dot * winv_ref[...]
    loc = loc_ref[0]
    val = val_ref[0]
    rows = lax.broadcasted_iota(jnp.int32, (_C1, _BS), 0)
    mask = rows == loc
    st = jnp.sum(jnp.where(mask, s, 0.0), axis=0, keepdims=True)
    t = st * (1.0 / _SCALE)
    sin_t = jnp.sqrt(jnp.clip(1.0 - t * t, 0.0, None))
    ctm = t * _COS_M - sin_t * _SIN_M
    ft64 = jnp.where(t > _THETA, ctm, t - _SINMM) * _SCALE
    s2 = jnp.where(mask, ft64, s)
    se = jnp.sum(jnp.exp(s2), axis=0, keepdims=True)
    part = jnp.sum((jnp.log(se) - ft64) * val) * (1.0 / _B)

    @pl.when(i == 0)
    def _():
        out_ref[...] = jnp.zeros((1, 1), jnp.float32)

    out_ref[...] = out_ref[...] + part


def _tc_grid_spec():
    return pltpu.PrefetchScalarGridSpec(
        num_scalar_prefetch=1,
        grid=(_NB,),
        in_specs=[
            pl.BlockSpec((_BS, _D), lambda i, gid: (i, 0)),
            pl.BlockSpec((1, _C1, _D), lambda i, gid: (gid[i], 0, 0)),
            pl.BlockSpec((1, 1, _BS), lambda i, gid: (i, 0, 0)),
            pl.BlockSpec((1, 1, _BS), lambda i, gid: (i, 0, 0)),
        ],
        out_specs=pl.BlockSpec((1, 1), lambda i, gid: (0, 0)),
        scratch_shapes=[pltpu.VMEM((_C1, 1), jnp.float32)],
    )


def kernel(logits, labels, weight):
    block_gid, src, rloc, valid = _routing(labels)
    xg = _sc_gather(logits, src)
    loc3 = rloc.reshape(_NB, 1, _BS)
    val3 = valid.astype(jnp.float32).reshape(_NB, 1, _BS)
    out = pl.pallas_call(
        _arc_step,
        grid_spec=_tc_grid_spec(),
        out_shape=jax.ShapeDtypeStruct((1, 1), jnp.float32),
    )(block_gid, xg, weight, loc3, val3)
    return out[0, 0]


# speedup vs baseline: 1.3144x; 1.2430x over previous
import functools
import math

import jax
import jax.numpy as jnp
from jax import lax
from jax.experimental import pallas as pl
from jax.experimental.pallas import tpu as pltpu
from jax.experimental.pallas import tpu_sc as plsc

_SCALE = 64.0
_MARGIN = 0.5
_COS_M = math.cos(_MARGIN)
_SIN_M = math.sin(_MARGIN)
_THETA = math.cos(math.pi - _MARGIN)
_SINMM = math.sin(math.pi - _MARGIN) * _MARGIN

_B = 4096
_C0 = 100
_C1 = 1000
_D = 512
_BS = 32
_NB = 232
_NPAD = _NB * _BS
_NW = 32
_BPW = _NPAD // _NW
_CH0, _CH1 = 120, 112
_EPS = 1e-12


def _routing(labels):
    labels = labels.astype(jnp.int32)
    group = labels // _C1
    local = labels % _C1
    oh = (group[:, None] == jnp.arange(_C0, dtype=jnp.int32)[None, :]).astype(jnp.int32)
    csum = jnp.cumsum(oh, axis=0)
    counts = csum[_B - 1]
    rank = jnp.take_along_axis(csum, group[:, None], axis=1)[:, 0] - 1
    bpg = (counts + _BS - 1) // _BS
    cum_bpg = jnp.cumsum(bpg)
    total_blocks = cum_bpg[_C0 - 1]
    block_start = cum_bpg - bpg
    bidx = jnp.arange(_NB, dtype=jnp.int32)
    block_gid = jnp.searchsorted(
        cum_bpg, jnp.minimum(bidx, total_blocks - 1), side="right"
    ).astype(jnp.int32)
    pos = block_start[group] * _BS + rank
    payload = jnp.stack(
        [jnp.arange(_B, dtype=jnp.int32), local, jnp.ones((_B,), jnp.int32)], axis=1)
    dest = jnp.zeros((_NPAD, 3), jnp.int32).at[pos].set(payload)
    src = dest[:, 0]
    rloc = dest[:, 1]
    valid = dest[:, 2] > 0
    return block_gid, src, rloc, valid


def _sc_gather(logits, src):
    mesh = plsc.VectorSubcoreMesh(core_axis_name="c", subcore_axis_name="s")

    @functools.partial(
        pl.kernel,
        out_type=jax.ShapeDtypeStruct((_NPAD, _D), jnp.float32),
        mesh=mesh,
        scratch_types=[
            pltpu.VMEM((_CH0,), jnp.int32),
            pltpu.VMEM((_CH1,), jnp.int32),
            pltpu.VMEM((_CH0, _D), jnp.float32),
            pltpu.VMEM((_CH1, _D), jnp.float32),
            pltpu.SemaphoreType.DMA,
            pltpu.SemaphoreType.DMA,
        ],
    )
    def k(logits_hbm, src_hbm, out_hbm, idx_a, idx_b, rows_a, rows_b, sem_a, sem_b):
        wid = lax.axis_index("s") * 2 + lax.axis_index("c")
        base = wid * _BPW
        pltpu.sync_copy(src_hbm.at[pl.ds(base, _CH0)], idx_a)
        pltpu.sync_copy(src_hbm.at[pl.ds(base + _CH0, _CH1)], idx_b)
        ga = pltpu.async_copy(logits_hbm.at[idx_a], rows_a, sem_a)
        gb = pltpu.async_copy(logits_hbm.at[idx_b], rows_b, sem_b)
        ga.wait()
        wa = pltpu.async_copy(rows_a, out_hbm.at[pl.ds(base, _CH0)], sem_a)
        gb.wait()
        wb = pltpu.async_copy(rows_b, out_hbm.at[pl.ds(base + _CH0, _CH1)], sem_b)
        wa.wait()
        wb.wait()

    return k(logits, src)


def _arc_step(gid_ref, xg_ref, w_ref, loc_ref, val_ref, out_ref, winv_ref):
    i = pl.program_id(0)
    g = gid_ref[i]
    prev = gid_ref[jnp.maximum(i - 1, 0)]
    w = w_ref[0]

    @pl.when((i == 0) | (g != prev))
    def _():
        ss = jnp.sum(w * w, axis=1, keepdims=True)
        winv_ref[...] = _SCALE / jnp.maximum(jnp.sqrt(ss), _EPS)

    x = xg_ref[...]
    xinv = 1.0 / jnp.maximum(
        jnp.sqrt(jnp.sum(x * x, axis=1, keepdims=True)), _EPS)
    xh = x * xinv
    dot = lax.dot_general(w, xh, (((1,), (1,)), ((), ())),
                          preferred_element_type=jnp.float32)
    s = dot * winv_ref[...]
    loc = loc_ref[0]
    val = val_ref[0]
    rows = lax.broadcasted_iota(jnp.int32, (_C1, _BS), 0)
    mask = rows == loc
    st = jnp.sum(jnp.where(mask, s, 0.0), axis=0, keepdims=True)
    t = st * (1.0 / _SCALE)
    sin_t = jnp.sqrt(jnp.clip(1.0 - t * t, 0.0, None))
    ctm = t * _COS_M - sin_t * _SIN_M
    ft64 = jnp.where(t > _THETA, ctm, t - _SINMM) * _SCALE
    s2 = jnp.where(mask, ft64, s)
    se = jnp.sum(jnp.exp(s2), axis=0, keepdims=True)
    part = jnp.sum((jnp.log(se) - ft64) * val) * (1.0 / _B)

    @pl.when(i == 0)
    def _():
        out_ref[...] = jnp.zeros((1, 1), jnp.float32)

    out_ref[...] = out_ref[...] + part


def _tc_grid_spec():
    return pltpu.PrefetchScalarGridSpec(
        num_scalar_prefetch=1,
        grid=(_NB,),
        in_specs=[
            pl.BlockSpec((_BS, _D), lambda i, gid: (i, 0)),
            pl.BlockSpec((1, _C1, _D), lambda i, gid: (gid[i], 0, 0)),
            pl.BlockSpec((1, 1, _BS), lambda i, gid: (i, 0, 0)),
            pl.BlockSpec((1, 1, _BS), lambda i, gid: (i, 0, 0)),
        ],
        out_specs=pl.BlockSpec((1, 1), lambda i, gid: (0, 0)),
        scratch_shapes=[pltpu.VMEM((_C1, 1), jnp.float32)],
    )


def kernel(logits, labels, weight):
    block_gid, src, rloc, valid = _routing(labels)
    xg = _sc_gather(logits, src)
    loc3 = rloc.reshape(_NB, 1, _BS)
    val3 = valid.astype(jnp.float32).reshape(_NB, 1, _BS)
    out = pl.pallas_call(
        _arc_step,
        grid_spec=_tc_grid_spec(),
        out_shape=jax.ShapeDtypeStruct((1, 1), jnp.float32),
    )(block_gid, xg, weight, loc3, val3)
    return out[0, 0]


# speedup vs baseline: 1.7352x; 1.3202x over previous
import functools
import math

import jax
import jax.numpy as jnp
from jax import lax
from jax.experimental import pallas as pl
from jax.experimental.pallas import tpu as pltpu
from jax.experimental.pallas import tpu_sc as plsc

_SCALE = 64.0
_MARGIN = 0.5
_COS_M = math.cos(_MARGIN)
_SIN_M = math.sin(_MARGIN)
_THETA = math.cos(math.pi - _MARGIN)
_SINMM = math.sin(math.pi - _MARGIN) * _MARGIN

_B = 4096
_C0 = 100
_C1 = 1000
_D = 512
_BS = 32
_NB = 232
_NPAD = _NB * _BS
_NW = 32
_BPW = _NPAD // _NW
_CH0, _CH1 = 120, 112
_EPS = 1e-12


def _routing(labels):
    labels = labels.astype(jnp.int32)
    group = labels // _C1
    local = labels % _C1
    oh = (group[:, None] == jnp.arange(_C0, dtype=jnp.int32)[None, :]).astype(jnp.int32)
    csum = jnp.cumsum(oh, axis=0)
    counts = csum[_B - 1]
    rank = jnp.take_along_axis(csum, group[:, None], axis=1)[:, 0] - 1
    bpg = (counts + _BS - 1) // _BS
    cum_bpg = jnp.cumsum(bpg)
    total_blocks = cum_bpg[_C0 - 1]
    block_start = cum_bpg - bpg
    bidx = jnp.arange(_NB, dtype=jnp.int32)
    block_gid = jnp.searchsorted(
        cum_bpg, jnp.minimum(bidx, total_blocks - 1), side="right"
    ).astype(jnp.int32)
    pos = block_start[group] * _BS + rank
    payload = jnp.stack([local, jnp.ones((_B,), jnp.int32)], axis=1)
    dest = jnp.zeros((_NPAD, 2), jnp.int32).at[pos].set(payload)
    rloc = dest[:, 0]
    valid = dest[:, 1] > 0
    return block_gid, pos, rloc, valid


def _sc_scatter(logits, pos):
    mesh = plsc.VectorSubcoreMesh(core_axis_name="c", subcore_axis_name="s")
    rows_w = _B // _NW

    @functools.partial(
        pl.kernel,
        out_type=jax.ShapeDtypeStruct((_NPAD, _D), jnp.float32),
        mesh=mesh,
        scratch_types=[
            pltpu.VMEM((rows_w,), jnp.int32),
            pltpu.VMEM((rows_w, _D), jnp.float32),
            pltpu.SemaphoreType.DMA,
        ],
    )
    def k(logits_hbm, pos_hbm, out_hbm, idx_v, rows_v, sem):
        wid = lax.axis_index("s") * 2 + lax.axis_index("c")
        base = wid * rows_w
        pltpu.sync_copy(pos_hbm.at[pl.ds(base, rows_w)], idx_v)
        pltpu.sync_copy(logits_hbm.at[pl.ds(base, rows_w)], rows_v)
        pltpu.async_copy(rows_v, out_hbm.at[idx_v], sem).wait()

    return k(logits, pos)


def _arc_step(gid_ref, xg_ref, w_ref, loc_ref, val_ref, out_ref, winv_ref):
    i = pl.program_id(0)
    g = gid_ref[i]
    prev = gid_ref[jnp.maximum(i - 1, 0)]
    w = w_ref[0]

    @pl.when((i == 0) | (g != prev))
    def _():
        ss = jnp.sum(w * w, axis=1, keepdims=True)
        winv_ref[...] = _SCALE / jnp.maximum(jnp.sqrt(ss), _EPS)

    x = xg_ref[...]
    xinv = 1.0 / jnp.maximum(
        jnp.sqrt(jnp.sum(x * x, axis=1, keepdims=True)), _EPS)
    xh = x * xinv
    dot = lax.dot_general(w, xh, (((1,), (1,)), ((), ())),
                          preferred_element_type=jnp.float32)
    s = dot * winv_ref[...]
    loc = loc_ref[0]
    val = val_ref[0]
    rows = lax.broadcasted_iota(jnp.int32, (_C1, _BS), 0)
    mask = rows == loc
    st = jnp.sum(jnp.where(mask, s, 0.0), axis=0, keepdims=True)
    t = st * (1.0 / _SCALE)
    sin_t = jnp.sqrt(jnp.clip(1.0 - t * t, 0.0, None))
    ctm = t * _COS_M - sin_t * _SIN_M
    ft64 = jnp.where(t > _THETA, ctm, t - _SINMM) * _SCALE
    s2 = jnp.where(mask, ft64, s)
    se = jnp.sum(jnp.exp(s2), axis=0, keepdims=True)
    part = jnp.sum(jnp.where(val > 0.0, jnp.log(se) - ft64, 0.0)) * (1.0 / _B)

    @pl.when(i == 0)
    def _():
        out_ref[...] = jnp.zeros((1, 1), jnp.float32)

    out_ref[...] = out_ref[...] + part


def _tc_grid_spec():
    return pltpu.PrefetchScalarGridSpec(
        num_scalar_prefetch=1,
        grid=(_NB,),
        in_specs=[
            pl.BlockSpec((_BS, _D), lambda i, gid: (i, 0)),
            pl.BlockSpec((1, _C1, _D), lambda i, gid: (gid[i], 0, 0)),
            pl.BlockSpec((1, 1, _BS), lambda i, gid: (i, 0, 0)),
            pl.BlockSpec((1, 1, _BS), lambda i, gid: (i, 0, 0)),
        ],
        out_specs=pl.BlockSpec((1, 1), lambda i, gid: (0, 0)),
        scratch_shapes=[pltpu.VMEM((_C1, 1), jnp.float32)],
    )


def kernel(logits, labels, weight):
    block_gid, pos, rloc, valid = _routing(labels)
    xg = _sc_scatter(logits, pos)
    loc3 = rloc.reshape(_NB, 1, _BS)
    val3 = valid.astype(jnp.float32).reshape(_NB, 1, _BS)
    out = pl.pallas_call(
        _arc_step,
        grid_spec=_tc_grid_spec(),
        out_shape=jax.ShapeDtypeStruct((1, 1), jnp.float32),
    )(block_gid, xg, weight, loc3, val3)
    return out[0, 0]


# speedup vs baseline: 1.7352x; 1.0000x over previous
import functools
import math

import jax
import jax.numpy as jnp
from jax import lax
from jax.experimental import pallas as pl
from jax.experimental.pallas import tpu as pltpu
from jax.experimental.pallas import tpu_sc as plsc

_SCALE = 64.0
_MARGIN = 0.5
_COS_M = math.cos(_MARGIN)
_SIN_M = math.sin(_MARGIN)
_THETA = math.cos(math.pi - _MARGIN)
_SINMM = math.sin(math.pi - _MARGIN) * _MARGIN

_B = 4096
_C0 = 100
_C1 = 1000
_D = 512
_BS = 32
_NB = 232
_NPAD = _NB * _BS
_NW = 32
_BPW = _NPAD // _NW
_CH0, _CH1 = 120, 112
_EPS = 1e-12


def _routing(labels):
    labels = labels.astype(jnp.int32)
    group = labels // _C1
    local = labels % _C1
    oh = (group[:, None] == jnp.arange(_C0, dtype=jnp.int32)[None, :]).astype(jnp.int32)
    csum = jnp.cumsum(oh, axis=0)
    counts = csum[_B - 1]
    rank = jnp.take_along_axis(
        csum, group[:, None], axis=1, mode="promise_in_bounds")[:, 0] - 1
    bpg = (counts + _BS - 1) // _BS
    cum_bpg = jnp.cumsum(bpg)
    total_blocks = cum_bpg[_C0 - 1]
    block_start = cum_bpg - bpg
    bidx = jnp.arange(_NB, dtype=jnp.int32)
    block_gid = jnp.searchsorted(
        cum_bpg, jnp.minimum(bidx, total_blocks - 1), side="right"
    ).astype(jnp.int32)
    pos = block_start[group] * _BS + rank
    payload = jnp.stack([local, jnp.ones((_B,), jnp.int32)], axis=1)
    dest = jnp.zeros((_NPAD, 2), jnp.int32).at[pos].set(
        payload, unique_indices=True, mode="promise_in_bounds")
    rloc = dest[:, 0]
    valid = dest[:, 1] > 0
    return block_gid, pos, rloc, valid


def _sc_scatter(logits, pos):
    mesh = plsc.VectorSubcoreMesh(core_axis_name="c", subcore_axis_name="s")
    rows_w = _B // _NW

    @functools.partial(
        pl.kernel,
        out_type=jax.ShapeDtypeStruct((_NPAD, _D), jnp.float32),
        mesh=mesh,
        scratch_types=[
            pltpu.VMEM((rows_w,), jnp.int32),
            pltpu.VMEM((rows_w, _D), jnp.float32),
            pltpu.SemaphoreType.DMA,
        ],
    )
    def k(logits_hbm, pos_hbm, out_hbm, idx_v, rows_v, sem):
        wid = lax.axis_index("s") * 2 + lax.axis_index("c")
        base = wid * rows_w
        pltpu.sync_copy(pos_hbm.at[pl.ds(base, rows_w)], idx_v)
        pltpu.sync_copy(logits_hbm.at[pl.ds(base, rows_w)], rows_v)
        pltpu.async_copy(rows_v, out_hbm.at[idx_v], sem).wait()

    return k(logits, pos)


def _arc_step(gid_ref, xg_ref, w_ref, loc_ref, val_ref, out_ref, winv_ref):
    i = pl.program_id(0)
    g = gid_ref[i]
    prev = gid_ref[jnp.maximum(i - 1, 0)]
    w = w_ref[0]

    @pl.when((i == 0) | (g != prev))
    def _():
        ss = jnp.sum(w * w, axis=1, keepdims=True)
        winv_ref[...] = _SCALE / jnp.maximum(jnp.sqrt(ss), _EPS)

    x = xg_ref[...]
    xinv = 1.0 / jnp.maximum(
        jnp.sqrt(jnp.sum(x * x, axis=1, keepdims=True)), _EPS)
    xh = x * xinv
    dot = lax.dot_general(w, xh, (((1,), (1,)), ((), ())),
                          preferred_element_type=jnp.float32)
    s = dot * winv_ref[...]
    loc = loc_ref[0]
    val = val_ref[0]
    rows = lax.broadcasted_iota(jnp.int32, (_C1, _BS), 0)
    mask = rows == loc
    st = jnp.sum(jnp.where(mask, s, 0.0), axis=0, keepdims=True)
    t = st * (1.0 / _SCALE)
    sin_t = jnp.sqrt(jnp.clip(1.0 - t * t, 0.0, None))
    ctm = t * _COS_M - sin_t * _SIN_M
    ft64 = jnp.where(t > _THETA, ctm, t - _SINMM) * _SCALE
    s2 = jnp.where(mask, ft64, s)
    se = jnp.sum(jnp.exp(s2), axis=0, keepdims=True)
    part = jnp.sum(jnp.where(val > 0.0, jnp.log(se) - ft64, 0.0)) * (1.0 / _B)

    @pl.when(i == 0)
    def _():
        out_ref[...] = jnp.zeros((1, 1), jnp.float32)

    out_ref[...] = out_ref[...] + part


def _tc_grid_spec():
    return pltpu.PrefetchScalarGridSpec(
        num_scalar_prefetch=1,
        grid=(_NB,),
        in_specs=[
            pl.BlockSpec((_BS, _D), lambda i, gid: (i, 0)),
            pl.BlockSpec((1, _C1, _D), lambda i, gid: (gid[i], 0, 0)),
            pl.BlockSpec((1, 1, _BS), lambda i, gid: (i, 0, 0)),
            pl.BlockSpec((1, 1, _BS), lambda i, gid: (i, 0, 0)),
        ],
        out_specs=pl.BlockSpec((1, 1), lambda i, gid: (0, 0)),
        scratch_shapes=[pltpu.VMEM((_C1, 1), jnp.float32)],
    )


def kernel(logits, labels, weight):
    block_gid, pos, rloc, valid = _routing(labels)
    xg = _sc_scatter(logits, pos)
    loc3 = rloc.reshape(_NB, 1, _BS)
    val3 = valid.astype(jnp.float32).reshape(_NB, 1, _BS)
    out = pl.pallas_call(
        _arc_step,
        grid_spec=_tc_grid_spec(),
        out_shape=jax.ShapeDtypeStruct((1, 1), jnp.float32),
    )(block_gid, xg, weight, loc3, val3)
    return out[0, 0]
